# baseline (speedup 1.0000x reference)
import functools

import jax
import jax.numpy as jnp
from jax.experimental import pallas as pl
from jax.experimental.pallas import tpu as pltpu

TOP_K = 2
NUM_EXPERTS = 8
TOKEN_TILE = 512


def _moe_dense_kernel(x_ref, wg_ref, bg_ref, ws_ref, be_ref, out_ref):
    x = x_ref[...]
    scores = jnp.dot(x, wg_ref[...], preferred_element_type=jnp.float32)
    scores = scores + bg_ref[...][None, :]
    m = jnp.max(scores, axis=-1, keepdims=True)
    ex = jnp.exp(scores - m)
    probs = ex / jnp.sum(ex, axis=-1, keepdims=True)
    lane = jax.lax.broadcasted_iota(jnp.int32, probs.shape, 1)
    i1 = jnp.argmax(probs, axis=-1, keepdims=True)
    mask1 = lane == i1
    neg = jnp.where(mask1, -jnp.inf, probs)
    i2 = jnp.argmax(neg, axis=-1, keepdims=True)
    mask2 = lane == i2
    cw = jnp.where(mask1 | mask2, probs, 0.0)

    xb = x.astype(jnp.bfloat16)
    cwb = cw.astype(jnp.bfloat16)
    xs = jnp.concatenate(
        [xb * cwb[:, e:e + 1] for e in range(NUM_EXPERTS)], axis=1)
    acc = jnp.dot(cw, be_ref[...], preferred_element_type=jnp.float32)
    acc = acc + jnp.dot(xs, ws_ref[...], preferred_element_type=jnp.float32)
    out_ref[...] = acc


@jax.jit
def kernel(x, Wg, bg, We, be):
    T, D = x.shape
    E, _, O = We.shape
    Ws_b = We.astype(jnp.bfloat16).reshape(E * D, O)
    grid = (T // TOKEN_TILE,)
    return pl.pallas_call(
        _moe_dense_kernel,
        grid=grid,
        in_specs=[
            pl.BlockSpec((TOKEN_TILE, D), lambda i: (i, 0)),
            pl.BlockSpec((D, E), lambda i: (0, 0)),
            pl.BlockSpec((E,), lambda i: (0,)),
            pl.BlockSpec((E * D, O), lambda i: (0, 0)),
            pl.BlockSpec((E, O), lambda i: (0, 0)),
        ],
        out_specs=pl.BlockSpec((TOKEN_TILE, O), lambda i: (i, 0)),
        out_shape=jax.ShapeDtypeStruct((T, O), jnp.float32),
        compiler_params=pltpu.CompilerParams(
            dimension_semantics=("arbitrary",),
        ),
    )(x, Wg, bg, Ws_b, be)
